# baseline (speedup 1.0000x reference)
import functools

import jax
import jax.numpy as jnp
from jax import lax
from jax.experimental import pallas as pl
from jax.experimental.pallas import tpu as pltpu
from jax.experimental.pallas import tpu_sc as plsc

N = 10000
E = 320000
H = 128
G = 64
NC = 2
NS = 16
NW = NC * NS

ECH = 80
EB = 128
NP = 10240
ZR = NP // NS

NQ = 4
QS = NP // NQ
QR = 2688
QSTR = QR // NS
GEB = 64
CNG = 40
EPAD = 327680
NCQ = EPAD // GEB
MAXCH = 8

XSR = 10112
XSTR = XSR // NS

ZCH = 4
ZB = 80

PCH = 125
PB = 80


def _mesh():
    return plsc.VectorSubcoreMesh(core_axis_name="c", subcore_axis_name="s",
                                  num_cores=NC, num_subcores=NS)


def _segment_sum(x, slists, dlists, counts, zeros):
    @functools.partial(
        pl.kernel,
        out_type=jax.ShapeDtypeStruct((NQ, QR, H), jnp.float32),
        mesh=_mesh(),
        scratch_types=[
            pltpu.VMEM((CNG, GEB), jnp.int32),
            pltpu.VMEM((CNG, GEB), jnp.int32),
            pltpu.VMEM((128,), jnp.int32),
            pltpu.VMEM((128,), jnp.int32),
            pltpu.VMEM((GEB, H), jnp.float32),
            pltpu.VMEM((GEB, H), jnp.float32),
            pltpu.VMEM_SHARED((XSR, H), jnp.float32),
            pltpu.VMEM_SHARED((QR, H), jnp.float32),
            pltpu.SemaphoreType.DMA,
            pltpu.SemaphoreType.DMA,
            pltpu.SemaphoreType.DMA,
            pltpu.SemaphoreType.DMA,
        ],
    )
    def k(x_hbm, sl_hbm, dl_hbm, cnt_hbm, zeros_hbm, out_hbm,
          sw_v, dw_v, cnt_v, cnt2_v, rows_a, rows_b, x_sh, acc_sh,
          ga, gb, sa, sb):
        cid = lax.axis_index("c")
        sid = lax.axis_index("s")
        xstripe = pl.ds(sid * XSTR, XSTR)
        pltpu.sync_copy(x_hbm.at[xstripe], x_sh.at[xstripe])
        astripe = pl.ds(sid * QSTR, QSTR)

        def gather(g, buf, sem):
            pltpu.async_copy(x_sh.at[sw_v.at[g]], buf, sem)

        def gwait(g, buf, sem):
            pltpu.make_async_copy(x_sh.at[sw_v.at[g]], buf, sem).wait()

        def scat(g, buf, sem):
            pltpu.async_copy(buf, acc_sh.at[dw_v.at[g]], sem, add=True)

        def swait(g, buf, sem):
            pltpu.make_async_copy(buf, acc_sh.at[dw_v.at[g]], sem).wait()

        for pp in range(2):
            q = 2 * cid + pp
            pltpu.sync_copy(zeros_hbm.at[pl.ds(0, QSTR)], acc_sh.at[astripe])
            plsc.subcore_barrier()

            pltpu.sync_copy(cnt_hbm.at[2 * q], cnt_v)
            pltpu.sync_copy(cnt_hbm.at[2 * q + 1], cnt2_v)
            ngrp = cnt_v[pl.ds(0, 16)][0]
            ngt = cnt2_v[pl.ds(0, 16)][0]
            gt0 = pl.multiple_of(sid * ngt, 8)
            tcnt = jnp.maximum(jnp.minimum(ngrp - gt0, ngt), 0)
            for ch in range(MAXCH):
                    ngr = jnp.maximum(jnp.minimum(tcnt - ch * CNG, CNG), 0)

                    @pl.when(ngr > 0)
                    def _():
                        pltpu.sync_copy(
                            sl_hbm.at[q, pl.ds(gt0 + ch * CNG, CNG)], sw_v)
                        pltpu.sync_copy(
                            dl_hbm.at[q, pl.ds(gt0 + ch * CNG, CNG)], dw_v)
                        gather(0, rows_a, ga)

                        @pl.when(ngr > 1)
                        def _():
                            gather(1, rows_b, gb)

                        def body(jj, carry):
                            g0 = 2 * jj
                            g1 = g0 + 1

                            @pl.when(g0 < ngr)
                            def _():
                                gwait(g0, rows_a, ga)
                                scat(g0, rows_a, sa)

                            @pl.when(g1 < ngr)
                            def _():
                                gwait(g1, rows_b, gb)
                                scat(g1, rows_b, sb)

                            @pl.when(g0 + 2 < ngr)
                            def _():
                                swait(g0, rows_a, sa)
                                gather(g0 + 2, rows_a, ga)

                            @pl.when(g1 + 2 < ngr)
                            def _():
                                swait(g1, rows_b, sb)
                                gather(g1 + 2, rows_b, gb)

                            return carry

                        lax.fori_loop(0, (ngr + 1) // 2, body, 0)
                        last_a = ((ngr + 1) // 2) * 2 - 2
                        last_b = (ngr // 2) * 2 - 1

                        @pl.when(last_a >= 0)
                        def _():
                            swait(last_a, rows_a, sa)

                        @pl.when(last_b >= 0)
                        def _():
                            swait(last_b, rows_b, sb)

            plsc.subcore_barrier()
            pltpu.sync_copy(acc_sh.at[astripe], out_hbm.at[q, astripe])
            if pp == 0:
                plsc.subcore_barrier()

    return k(x, slists, dlists, counts, zeros)


def _emb_lookup(z_pad, table):
    @functools.partial(
        pl.kernel,
        out_type=jax.ShapeDtypeStruct((NP, H), jnp.float32),
        mesh=_mesh(),
        scratch_types=[
            pltpu.VMEM((ZCH, ZB), jnp.int32),
            pltpu.VMEM((ZB, H), jnp.float32),
        ],
    )
    def k(z_hbm, tab_hbm, out_hbm, z_v, rows_v):
        cid = lax.axis_index("c")
        sid = lax.axis_index("s")
        wid = sid * NC + cid
        pltpu.sync_copy(z_hbm.at[wid], z_v)
        for j in range(ZCH):
            pltpu.sync_copy(tab_hbm.at[z_v.at[j]], rows_v)
            pltpu.sync_copy(rows_v, out_hbm.at[pl.ds(wid * ZCH * ZB + j * ZB, ZB)])

    return k(z_pad, table)


def _pool(x1, x2, x3, batch_r, zeros, ones):
    out_t = jax.ShapeDtypeStruct((NC, G, H), jnp.float32)

    @functools.partial(
        pl.kernel,
        out_type=[out_t, out_t, out_t, out_t],
        mesh=_mesh(),
        scratch_types=[
            pltpu.VMEM((PB,), jnp.int32),
            pltpu.VMEM((PB, H), jnp.float32),
            pltpu.VMEM((PB, H), jnp.float32),
            pltpu.VMEM_SHARED((G, H), jnp.float32),
            pltpu.VMEM_SHARED((G, H), jnp.float32),
            pltpu.VMEM_SHARED((G, H), jnp.float32),
            pltpu.VMEM_SHARED((G, H), jnp.float32),
        ],
    )
    def k(x1_hbm, x2_hbm, x3_hbm, b_hbm, zeros_hbm, ones_hbm,
          o1, o2, o3, oc, bidx_v, rows_v, ones_v, a1, a2, a3, ac):
        cid = lax.axis_index("c")
        sid = lax.axis_index("s")
        wid = sid * NC + cid

        @pl.when(sid == 0)
        def _():
            for a in (a1, a2, a3, ac):
                pltpu.sync_copy(zeros_hbm.at[pl.ds(0, G)], a)

        pltpu.sync_copy(ones_hbm, ones_v)
        plsc.subcore_barrier()

        def body(kk, carry):
            ch = wid + NW * kk

            @pl.when(ch < PCH)
            def _():
                base = ch * PB
                pltpu.sync_copy(b_hbm.at[ch], bidx_v)
                pltpu.sync_copy(x1_hbm.at[pl.ds(base, PB)], rows_v)
                pltpu.sync_copy(rows_v, a1.at[bidx_v], add=True)
                pltpu.sync_copy(x2_hbm.at[pl.ds(base, PB)], rows_v)
                pltpu.sync_copy(rows_v, a2.at[bidx_v], add=True)
                pltpu.sync_copy(x3_hbm.at[pl.ds(base, PB)], rows_v)
                pltpu.sync_copy(rows_v, a3.at[bidx_v], add=True)
                pltpu.sync_copy(ones_v, ac.at[bidx_v], add=True)

            return carry

        lax.fori_loop(0, (PCH + NW - 1) // NW, body, 0)
        plsc.subcore_barrier()

        @pl.when(sid == 0)
        def _():
            pltpu.sync_copy(a1, o1.at[cid])
            pltpu.sync_copy(a2, o2.at[cid])
            pltpu.sync_copy(a3, o3.at[cid])
            pltpu.sync_copy(ac, oc.at[cid])

    return k(x1, x2, x3, batch_r, zeros, ones)


_R = 1000
_NB = N // _R


def _mlp_bn_kernel(x_ref, p_ref, w1_ref, b1_ref, w2_ref, b2_ref, g_ref, bt_ref,
                   out_ref, h2_buf, s_ref, ss_ref):
    p = pl.program_id(0)
    i = pl.program_id(1)

    @pl.when(p == 0)
    def _():
        h0 = x_ref[...] + p_ref[...]
        h1 = jnp.maximum(
            jnp.dot(h0, w1_ref[...], preferred_element_type=jnp.float32)
            + b1_ref[...], 0.0)
        h2 = jnp.maximum(
            jnp.dot(h1, w2_ref[...], preferred_element_type=jnp.float32)
            + b2_ref[...], 0.0)
        h2_buf[pl.ds(i * _R, _R), :] = h2

        @pl.when(i == 0)
        def _():
            s_ref[...] = jnp.zeros_like(s_ref)
            ss_ref[...] = jnp.zeros_like(ss_ref)

        s_ref[...] += jnp.sum(h2, axis=0, keepdims=True)
        ss_ref[...] += jnp.sum(h2 * h2, axis=0, keepdims=True)

    @pl.when(p == 1)
    def _():
        h2 = h2_buf[pl.ds(i * _R, _R), :]
        mu = s_ref[...] * (1.0 / N)
        var = ss_ref[...] * (1.0 / N) - mu * mu
        inv = g_ref[...] * lax.rsqrt(var + 1e-5)
        out_ref[...] = h2 * inv + (bt_ref[...] - mu * inv)


def _mlp_bn(x, agg, W1, b1, W2, b2, g, bt):
    row = lambda a: a.reshape(1, H)
    return pl.pallas_call(
        _mlp_bn_kernel,
        grid=(2, _NB),
        in_specs=[
            pl.BlockSpec((_R, H), lambda p, i: (i * (1 - p), 0)),
            pl.BlockSpec((_R, H), lambda p, i: (i * (1 - p), 0)),
            pl.BlockSpec((H, H), lambda p, i: (0, 0)),
            pl.BlockSpec((1, H), lambda p, i: (0, 0)),
            pl.BlockSpec((H, H), lambda p, i: (0, 0)),
            pl.BlockSpec((1, H), lambda p, i: (0, 0)),
            pl.BlockSpec((1, H), lambda p, i: (0, 0)),
            pl.BlockSpec((1, H), lambda p, i: (0, 0)),
        ],
        out_specs=pl.BlockSpec((_R, H), lambda p, i: (i, 0)),
        out_shape=jax.ShapeDtypeStruct((NP, H), jnp.float32),
        scratch_shapes=[
            pltpu.VMEM((N, H), jnp.float32),
            pltpu.VMEM((1, H), jnp.float32),
            pltpu.VMEM((1, H), jnp.float32),
        ],
    )(x, agg, W1, row(b1), W2, row(b2), row(g), row(bt))


def _head_kernel(s1, s2, s3, cn, w1a, w1b, w1c, b1, w2, b2, out_ref):
    cnt = jnp.maximum(cn[0, :, :1] + cn[1, :, :1], 1.0)
    p1 = (s1[0] + s1[1]) / cnt
    p2 = (s2[0] + s2[1]) / cnt
    p3 = (s3[0] + s3[1]) / cnt
    h = jnp.maximum(
        jnp.dot(p1, w1a[...], preferred_element_type=jnp.float32)
        + jnp.dot(p2, w1b[...], preferred_element_type=jnp.float32)
        + jnp.dot(p3, w1c[...], preferred_element_type=jnp.float32)
        + b1[...], 0.0)
    o = jnp.sum(h * w2[...], axis=1, keepdims=True) + b2[0, :1]
    out_ref[...] = jnp.broadcast_to(o, (G, H))


def _head(s1, s2, s3, cn, lin1_W, lin1_b, lin2_W, lin2_b):
    pspec = pl.BlockSpec((NC, G, H), lambda: (0, 0, 0))
    wspec = pl.BlockSpec((H, H), lambda: (0, 0))
    vspec = pl.BlockSpec((1, H), lambda: (0, 0))
    out = pl.pallas_call(
        _head_kernel,
        in_specs=[pspec] * 4 + [wspec] * 3 + [vspec] * 3,
        out_specs=pl.BlockSpec((G, H), lambda: (0, 0)),
        out_shape=jax.ShapeDtypeStruct((G, H), jnp.float32),
    )(s1, s2, s3, cn,
      lin1_W[0:H], lin1_W[H:2 * H], lin1_W[2 * H:3 * H],
      lin1_b.reshape(1, H), lin2_W.reshape(1, H),
      jnp.broadcast_to(lin2_b.reshape(1, 1), (1, H)))
    return out[:, :1]


def kernel(z, edge_index, batch, z_emb_table,
           W1_0, b1_0, W2_0, b2_0, g_0, bt_0,
           W1_1, b1_1, W2_1, b2_1, g_1, bt_1,
           W1_2, b1_2, W2_2, b2_2, g_2, bt_2,
           lin1_W, lin1_b, lin2_W, lin2_b):
    srcp = jnp.concatenate([edge_index[0].astype(jnp.int32),
                            jnp.zeros((EPAD - E,), jnp.int32)])
    dstp = jnp.concatenate([edge_index[1].astype(jnp.int32),
                            jnp.full((EPAD - E,), N, jnp.int32)])
    qkey = dstp // QS
    dloc = dstp - qkey * QS
    masks = qkey[None, :] == jnp.arange(NQ)[:, None]
    ranks = jnp.cumsum(masks.astype(jnp.int32), axis=1)
    qcap = NCQ * GEB
    dest = jnp.sum(jnp.where(masks,
                             jnp.arange(NQ)[:, None] * qcap + ranks - 1,
                             0), axis=0).astype(jnp.int32)
    slists = jnp.zeros((NQ * qcap,), jnp.int32).at[dest].set(srcp)
    dlists = jnp.full((NQ * qcap,), QS, jnp.int32).at[dest].set(dloc)
    slists = slists.reshape(NQ, NCQ, GEB)
    dlists = dlists.reshape(NQ, NCQ, GEB)
    ng_e = ranks[:, -1]
    ngrp = (ng_e + GEB - 1) // GEB
    ngt = ((ngrp + NS - 1) // NS + 7) // 8 * 8
    counts = jnp.broadcast_to(
        jnp.stack([ngrp, ngt], axis=1).reshape(NQ * 2, 1),
        (NQ * 2, 128)).astype(jnp.int32)

    z_pad = jnp.pad(z.astype(jnp.int32), (0, NP - N)).reshape(NW, ZCH, ZB)
    batch_r = batch.astype(jnp.int32).reshape(PCH, PB)
    zeros = jnp.zeros((ZR, H), jnp.float32)
    ones = jnp.ones((PB, H), jnp.float32)

    params = [(W1_0, b1_0, W2_0, b2_0, g_0, bt_0),
              (W1_1, b1_1, W2_1, b2_1, g_1, bt_1),
              (W1_2, b1_2, W2_2, b2_2, g_2, bt_2)]

    x = _emb_lookup(z_pad, z_emb_table)
    xs = []
    for p in params:
        partials = _segment_sum(x, slists, dlists, counts, zeros)
        agg = partials[:, :QS, :].reshape(NQ * QS, H)
        x = _mlp_bn(x, agg, *p)
        xs.append(x)

    s1, s2, s3, cn = _pool(xs[0], xs[1], xs[2], batch_r, zeros, ones)
    return _head(s1, s2, s3, cn, lin1_W, lin1_b, lin2_W, lin2_b)


# speedup vs baseline: 3.8604x; 3.8604x over previous
import functools

import jax
import jax.numpy as jnp
from jax import lax
from jax.experimental import pallas as pl
from jax.experimental.pallas import tpu as pltpu
from jax.experimental.pallas import tpu_sc as plsc

N = 10000
E = 320000
H = 128
G = 64
NC = 2
NS = 16
NW = NC * NS

ECH = 80
EB = 128
NP = 10240
ZR = NP // NS

NQ = 4
QS = NP // NQ
QR = 2688
QSTR = QR // NS
GEB = 64
CNG = 40
EPAD = 327680
NCQ = EPAD // GEB
MAXCH = 8

XSR = 10112
XSTR = XSR // NS

ZCH = 4
ZB = 80

PCH = 125
PB = 80


def _mesh():
    return plsc.VectorSubcoreMesh(core_axis_name="c", subcore_axis_name="s",
                                  num_cores=NC, num_subcores=NS)


def _segment_sum(x, slists, dlists, counts, zeros):
    @functools.partial(
        pl.kernel,
        out_type=jax.ShapeDtypeStruct((NQ, QR, H), jnp.float32),
        mesh=_mesh(),
        scratch_types=[
            pltpu.VMEM((CNG, GEB), jnp.int32),
            pltpu.VMEM((CNG, GEB), jnp.int32),
            pltpu.VMEM((128,), jnp.int32),
            pltpu.VMEM((128,), jnp.int32),
            pltpu.VMEM((GEB, H), jnp.float32),
            pltpu.VMEM((GEB, H), jnp.float32),
            pltpu.VMEM_SHARED((XSR, H), jnp.float32),
            pltpu.VMEM_SHARED((QR, H), jnp.float32),
            pltpu.SemaphoreType.DMA,
            pltpu.SemaphoreType.DMA,
            pltpu.SemaphoreType.DMA,
            pltpu.SemaphoreType.DMA,
        ],
    )
    def k(x_hbm, sl_hbm, dl_hbm, cnt_hbm, zeros_hbm, out_hbm,
          sw_v, dw_v, cnt_v, cnt2_v, rows_a, rows_b, x_sh, acc_sh,
          ga, gb, sa, sb):
        cid = lax.axis_index("c")
        sid = lax.axis_index("s")
        xstripe = pl.ds(sid * XSTR, XSTR)
        pltpu.sync_copy(x_hbm.at[xstripe], x_sh.at[xstripe])
        astripe = pl.ds(sid * QSTR, QSTR)

        def gather(g, buf, sem):
            pltpu.async_copy(x_sh.at[sw_v.at[g]], buf, sem)

        def gwait(g, buf, sem):
            pltpu.make_async_copy(x_sh.at[sw_v.at[g]], buf, sem).wait()

        def scat(g, buf, sem):
            pltpu.async_copy(buf, acc_sh.at[dw_v.at[g]], sem, add=True)

        def swait(g, buf, sem):
            pltpu.make_async_copy(buf, acc_sh.at[dw_v.at[g]], sem).wait()

        for pp in range(2):
            q = 2 * cid + pp
            pltpu.sync_copy(zeros_hbm.at[pl.ds(0, QSTR)], acc_sh.at[astripe])
            plsc.subcore_barrier()

            pltpu.sync_copy(cnt_hbm.at[2 * q], cnt_v)
            pltpu.sync_copy(cnt_hbm.at[2 * q + 1], cnt2_v)
            ngrp = cnt_v[pl.ds(0, 16)][0]
            ngt = cnt2_v[pl.ds(0, 16)][0]
            gt0 = pl.multiple_of(sid * ngt, 8)
            tcnt = jnp.maximum(jnp.minimum(ngrp - gt0, ngt), 0)
            for ch in range(MAXCH):
                    ngr = jnp.maximum(jnp.minimum(tcnt - ch * CNG, CNG), 0)

                    @pl.when(ngr > 0)
                    def _():
                        pltpu.sync_copy(
                            sl_hbm.at[q, pl.ds(gt0 + ch * CNG, CNG)], sw_v)
                        pltpu.sync_copy(
                            dl_hbm.at[q, pl.ds(gt0 + ch * CNG, CNG)], dw_v)
                        gather(0, rows_a, ga)

                        @pl.when(ngr > 1)
                        def _():
                            gather(1, rows_b, gb)

                        def body(jj, carry):
                            g0 = 2 * jj
                            g1 = g0 + 1

                            @pl.when(g0 < ngr)
                            def _():
                                gwait(g0, rows_a, ga)
                                scat(g0, rows_a, sa)

                            @pl.when(g1 < ngr)
                            def _():
                                gwait(g1, rows_b, gb)
                                scat(g1, rows_b, sb)

                            @pl.when(g0 + 2 < ngr)
                            def _():
                                swait(g0, rows_a, sa)
                                gather(g0 + 2, rows_a, ga)

                            @pl.when(g1 + 2 < ngr)
                            def _():
                                swait(g1, rows_b, sb)
                                gather(g1 + 2, rows_b, gb)

                            return carry

                        lax.fori_loop(0, (ngr + 1) // 2, body, 0)
                        last_a = ((ngr + 1) // 2) * 2 - 2
                        last_b = (ngr // 2) * 2 - 1

                        @pl.when(last_a >= 0)
                        def _():
                            swait(last_a, rows_a, sa)

                        @pl.when(last_b >= 0)
                        def _():
                            swait(last_b, rows_b, sb)

            plsc.subcore_barrier()
            pltpu.sync_copy(acc_sh.at[astripe], out_hbm.at[q, astripe])
            if pp == 0:
                plsc.subcore_barrier()

    return k(x, slists, dlists, counts, zeros)


def _emb_lookup(z_pad, table):
    @functools.partial(
        pl.kernel,
        out_type=jax.ShapeDtypeStruct((NP, H), jnp.float32),
        mesh=_mesh(),
        scratch_types=[
            pltpu.VMEM((ZCH, ZB), jnp.int32),
            pltpu.VMEM((ZB, H), jnp.float32),
        ],
    )
    def k(z_hbm, tab_hbm, out_hbm, z_v, rows_v):
        cid = lax.axis_index("c")
        sid = lax.axis_index("s")
        wid = sid * NC + cid
        pltpu.sync_copy(z_hbm.at[wid], z_v)
        for j in range(ZCH):
            pltpu.sync_copy(tab_hbm.at[z_v.at[j]], rows_v)
            pltpu.sync_copy(rows_v, out_hbm.at[pl.ds(wid * ZCH * ZB + j * ZB, ZB)])

    return k(z_pad, table)


def _pool(x1, x2, x3, batch_r, zeros, ones):
    out_t = jax.ShapeDtypeStruct((NC, G, H), jnp.float32)

    @functools.partial(
        pl.kernel,
        out_type=[out_t, out_t, out_t, out_t],
        mesh=_mesh(),
        scratch_types=[
            pltpu.VMEM((PB,), jnp.int32),
            pltpu.VMEM((PB, H), jnp.float32),
            pltpu.VMEM((PB, H), jnp.float32),
            pltpu.VMEM_SHARED((G, H), jnp.float32),
            pltpu.VMEM_SHARED((G, H), jnp.float32),
            pltpu.VMEM_SHARED((G, H), jnp.float32),
            pltpu.VMEM_SHARED((G, H), jnp.float32),
        ],
    )
    def k(x1_hbm, x2_hbm, x3_hbm, b_hbm, zeros_hbm, ones_hbm,
          o1, o2, o3, oc, bidx_v, rows_v, ones_v, a1, a2, a3, ac):
        cid = lax.axis_index("c")
        sid = lax.axis_index("s")
        wid = sid * NC + cid

        @pl.when(sid == 0)
        def _():
            for a in (a1, a2, a3, ac):
                pltpu.sync_copy(zeros_hbm.at[pl.ds(0, G)], a)

        pltpu.sync_copy(ones_hbm, ones_v)
        plsc.subcore_barrier()

        def body(kk, carry):
            ch = wid + NW * kk

            @pl.when(ch < PCH)
            def _():
                base = ch * PB
                pltpu.sync_copy(b_hbm.at[ch], bidx_v)
                pltpu.sync_copy(x1_hbm.at[pl.ds(base, PB)], rows_v)
                pltpu.sync_copy(rows_v, a1.at[bidx_v], add=True)
                pltpu.sync_copy(x2_hbm.at[pl.ds(base, PB)], rows_v)
                pltpu.sync_copy(rows_v, a2.at[bidx_v], add=True)
                pltpu.sync_copy(x3_hbm.at[pl.ds(base, PB)], rows_v)
                pltpu.sync_copy(rows_v, a3.at[bidx_v], add=True)
                pltpu.sync_copy(ones_v, ac.at[bidx_v], add=True)

            return carry

        lax.fori_loop(0, (PCH + NW - 1) // NW, body, 0)
        plsc.subcore_barrier()

        @pl.when(sid == 0)
        def _():
            pltpu.sync_copy(a1, o1.at[cid])
            pltpu.sync_copy(a2, o2.at[cid])
            pltpu.sync_copy(a3, o3.at[cid])
            pltpu.sync_copy(ac, oc.at[cid])

    return k(x1, x2, x3, batch_r, zeros, ones)


_R = 1000
_NB = N // _R


def _mlp_bn_kernel(x_ref, p_ref, w1_ref, b1_ref, w2_ref, b2_ref, g_ref, bt_ref,
                   out_ref, h2_buf, s_ref, ss_ref):
    p = pl.program_id(0)
    i = pl.program_id(1)

    @pl.when(p == 0)
    def _():
        h0 = x_ref[...] + p_ref[...]
        h1 = jnp.maximum(
            jnp.dot(h0, w1_ref[...], preferred_element_type=jnp.float32)
            + b1_ref[...], 0.0)
        h2 = jnp.maximum(
            jnp.dot(h1, w2_ref[...], preferred_element_type=jnp.float32)
            + b2_ref[...], 0.0)
        h2_buf[pl.ds(i * _R, _R), :] = h2

        @pl.when(i == 0)
        def _():
            s_ref[...] = jnp.zeros_like(s_ref)
            ss_ref[...] = jnp.zeros_like(ss_ref)

        s_ref[...] += jnp.sum(h2, axis=0, keepdims=True)
        ss_ref[...] += jnp.sum(h2 * h2, axis=0, keepdims=True)

    @pl.when(p == 1)
    def _():
        h2 = h2_buf[pl.ds(i * _R, _R), :]
        mu = s_ref[...] * (1.0 / N)
        var = ss_ref[...] * (1.0 / N) - mu * mu
        inv = g_ref[...] * lax.rsqrt(var + 1e-5)
        out_ref[...] = h2 * inv + (bt_ref[...] - mu * inv)


def _mlp_bn(x, agg, W1, b1, W2, b2, g, bt):
    row = lambda a: a.reshape(1, H)
    return pl.pallas_call(
        _mlp_bn_kernel,
        grid=(2, _NB),
        in_specs=[
            pl.BlockSpec((_R, H), lambda p, i: (i * (1 - p), 0)),
            pl.BlockSpec((_R, H), lambda p, i: (i * (1 - p), 0)),
            pl.BlockSpec((H, H), lambda p, i: (0, 0)),
            pl.BlockSpec((1, H), lambda p, i: (0, 0)),
            pl.BlockSpec((H, H), lambda p, i: (0, 0)),
            pl.BlockSpec((1, H), lambda p, i: (0, 0)),
            pl.BlockSpec((1, H), lambda p, i: (0, 0)),
            pl.BlockSpec((1, H), lambda p, i: (0, 0)),
        ],
        out_specs=pl.BlockSpec((_R, H), lambda p, i: (i, 0)),
        out_shape=jax.ShapeDtypeStruct((NP, H), jnp.float32),
        scratch_shapes=[
            pltpu.VMEM((N, H), jnp.float32),
            pltpu.VMEM((1, H), jnp.float32),
            pltpu.VMEM((1, H), jnp.float32),
        ],
    )(x, agg, W1, row(b1), W2, row(b2), row(g), row(bt))


def _head_kernel(s1, s2, s3, cn, w1a, w1b, w1c, b1, w2, b2, out_ref):
    cnt = jnp.maximum(cn[0, :, :1] + cn[1, :, :1], 1.0)
    p1 = (s1[0] + s1[1]) / cnt
    p2 = (s2[0] + s2[1]) / cnt
    p3 = (s3[0] + s3[1]) / cnt
    h = jnp.maximum(
        jnp.dot(p1, w1a[...], preferred_element_type=jnp.float32)
        + jnp.dot(p2, w1b[...], preferred_element_type=jnp.float32)
        + jnp.dot(p3, w1c[...], preferred_element_type=jnp.float32)
        + b1[...], 0.0)
    o = jnp.sum(h * w2[...], axis=1, keepdims=True) + b2[0, :1]
    out_ref[...] = jnp.broadcast_to(o, (G, H))


def _head(s1, s2, s3, cn, lin1_W, lin1_b, lin2_W, lin2_b):
    pspec = pl.BlockSpec((NC, G, H), lambda: (0, 0, 0))
    wspec = pl.BlockSpec((H, H), lambda: (0, 0))
    vspec = pl.BlockSpec((1, H), lambda: (0, 0))
    out = pl.pallas_call(
        _head_kernel,
        in_specs=[pspec] * 4 + [wspec] * 3 + [vspec] * 3,
        out_specs=pl.BlockSpec((G, H), lambda: (0, 0)),
        out_shape=jax.ShapeDtypeStruct((G, H), jnp.float32),
    )(s1, s2, s3, cn,
      lin1_W[0:H], lin1_W[H:2 * H], lin1_W[2 * H:3 * H],
      lin1_b.reshape(1, H), lin2_W.reshape(1, H),
      jnp.broadcast_to(lin2_b.reshape(1, 1), (1, H)))
    return out[:, :1]


def kernel(z, edge_index, batch, z_emb_table,
           W1_0, b1_0, W2_0, b2_0, g_0, bt_0,
           W1_1, b1_1, W2_1, b2_1, g_1, bt_1,
           W1_2, b1_2, W2_2, b2_2, g_2, bt_2,
           lin1_W, lin1_b, lin2_W, lin2_b):
    srcp = jnp.concatenate([edge_index[0].astype(jnp.int32),
                            jnp.zeros((EPAD - E,), jnp.int32)])
    dstp = jnp.concatenate([edge_index[1].astype(jnp.int32),
                            jnp.full((EPAD - E,), N, jnp.int32)])
    qkey = dstp // QS
    dloc = dstp - qkey * QS
    masks = qkey[None, :] == jnp.arange(NQ)[:, None]
    ranks = jnp.cumsum(masks.astype(jnp.int32), axis=1)
    qcap = NCQ * GEB
    dest = jnp.sum(jnp.where(masks,
                             jnp.arange(NQ)[:, None] * qcap + ranks - 1,
                             0), axis=0).astype(jnp.int32)
    slists = jnp.zeros((NQ * qcap,), jnp.int32).at[dest].add(
        srcp, unique_indices=True, mode="promise_in_bounds")
    dlists = QS + jnp.zeros((NQ * qcap,), jnp.int32).at[dest].add(
        dloc - QS, unique_indices=True, mode="promise_in_bounds")
    slists = slists.reshape(NQ, NCQ, GEB)
    dlists = dlists.reshape(NQ, NCQ, GEB)
    ng_e = ranks[:, -1]
    ngrp = (ng_e + GEB - 1) // GEB
    ngt = ((ngrp + NS - 1) // NS + 7) // 8 * 8
    counts = jnp.broadcast_to(
        jnp.stack([ngrp, ngt], axis=1).reshape(NQ * 2, 1),
        (NQ * 2, 128)).astype(jnp.int32)

    z_pad = jnp.pad(z.astype(jnp.int32), (0, NP - N)).reshape(NW, ZCH, ZB)
    batch_r = batch.astype(jnp.int32).reshape(PCH, PB)
    zeros = jnp.zeros((ZR, H), jnp.float32)
    ones = jnp.ones((PB, H), jnp.float32)

    params = [(W1_0, b1_0, W2_0, b2_0, g_0, bt_0),
              (W1_1, b1_1, W2_1, b2_1, g_1, bt_1),
              (W1_2, b1_2, W2_2, b2_2, g_2, bt_2)]

    x = _emb_lookup(z_pad, z_emb_table)
    xs = []
    for p in params:
        partials = _segment_sum(x, slists, dlists, counts, zeros)
        agg = partials[:, :QS, :].reshape(NQ * QS, H)
        x = _mlp_bn(x, agg, *p)
        xs.append(x)

    s1, s2, s3, cn = _pool(xs[0], xs[1], xs[2], batch_r, zeros, ones)
    return _head(s1, s2, s3, cn, lin1_W, lin1_b, lin2_W, lin2_b)
